# Initial kernel scaffold; baseline (speedup 1.0000x reference)
import jax
import jax.numpy as jnp
from jax.experimental import pallas as pl


def kernel(outputs, targets, empty_vec):
    raise NotImplementedError("write your pallas kernel here")


# speedup vs baseline: 56.2696x; 56.2696x over previous
import dataclasses
import functools

import jax
import jax.numpy as jnp
from jax import lax
from jax.experimental import pallas as pl
from jax.experimental.pallas import tpu as pltpu
from jax.experimental.pallas import tpu_sc as plsc

_B = 32
_Q = 300
_T = 40
_D = 512
_MP = 304
_NCH = _MP // 16
_EOS = 0.1
_BIGF = 3e38
_PADC = 1e30


def _tc_body(out_ref, tgt_ref, emp_ref, cost_ref, cose_ref):
    out = out_ref[0]
    tgt = tgt_ref[0]
    emp = emp_ref[...]
    on = out / jnp.maximum(
        jnp.sqrt(jnp.sum(out * out, axis=-1, keepdims=True)), 1e-12)
    tn = tgt / jnp.maximum(
        jnp.sqrt(jnp.sum(tgt * tgt, axis=-1, keepdims=True)), 1e-12)
    en = emp / jnp.maximum(
        jnp.sqrt(jnp.sum(emp * emp, axis=-1, keepdims=True)), 1e-12)
    cost = -lax.dot_general(tn, on, (((1,), (1,)), ((), ())),
                            precision=lax.Precision.HIGHEST,
                            preferred_element_type=jnp.float32)
    cose = lax.dot_general(en, on, (((1,), (1,)), ((), ())),
                           precision=lax.Precision.HIGHEST,
                           preferred_element_type=jnp.float32)
    cost_ref[0] = cost
    cose_ref[0] = cose


def _tc_cost(outputs, targets, empty_vec):
    return pl.pallas_call(
        _tc_body,
        grid=(_B,),
        in_specs=[
            pl.BlockSpec((1, _Q, _D), lambda b: (b, 0, 0)),
            pl.BlockSpec((1, _T, _D), lambda b: (b, 0, 0)),
            pl.BlockSpec((1, _D), lambda b: (0, 0)),
        ],
        out_specs=[
            pl.BlockSpec((1, _T, _Q), lambda b: (b, 0, 0)),
            pl.BlockSpec((1, 1, _Q), lambda b: (b, 0, 0)),
        ],
        out_shape=[
            jax.ShapeDtypeStruct((_B, _T, _Q), jnp.float32),
            jax.ShapeDtypeStruct((_B, 1, _Q), jnp.float32),
        ],
        compiler_params=pltpu.CompilerParams(
            dimension_semantics=("arbitrary",)),
    )(outputs, targets, empty_vec)


def _splat_i(x):
    return jnp.full((16,), x, jnp.int32)


def _gather1(ref, idxs):
    return jnp.min(plsc.load_gather(ref, [_splat_i(i) for i in idxs]))


def _sc_body(cost_hbm, cose_hbm, out_hbm,
             c_v, v_v, dist_v, way_v, scan_v, cose_v, stage_v,
             p_s, r2c_s, sem):
    wid = lax.axis_index("s") * 2 + lax.axis_index("c")
    pltpu.async_copy(cost_hbm.at[wid], c_v, sem).wait()
    pltpu.async_copy(cose_hbm.at[wid], cose_v, sem).wait()

    lane = lax.iota(jnp.int32, 16)
    lane0 = lane == 0

    @pl.loop(0, _MP)
    def _(j):
        p_s[j] = jnp.int32(-1)

    @pl.loop(0, _NCH)
    def _(ci):
        v_v[pl.ds(ci * 16, 16)] = jnp.zeros((16,), jnp.float32)

    def s0_chunk(ci, acc):
        return acc + jnp.sum(1.0 - cose_v[pl.ds(ci * 16, 16)])
    s0 = lax.fori_loop(0, _NCH, s0_chunk, jnp.float32(0.0))

    def row_phase(i, _carry):
        def init_chunk(ci, carry):
            bv, bj = carry
            base = ci * 16
            sl = pl.ds(base, 16)
            d = c_v[i, sl] - v_v[sl]
            dist_v[sl] = d
            way_v[sl] = _splat_i(-2)
            scan_v[sl] = jnp.zeros((16,), jnp.float32)
            m = jnp.min(d)
            mi = jnp.min(jnp.where(d == m, lane + base, jnp.int32(1 << 30)))
            better = m < bv
            return jnp.where(better, m, bv), jnp.where(better, mi, bj)

        bv, bj = lax.fori_loop(0, _NCH, init_chunk, (jnp.float32(_BIGF), jnp.int32(0)))

        def dcond(carry):
            jstar, _d = carry
            return p_s[jstar] != -1

        def dbody(carry):
            jstar, dmin = carry
            r = p_s[jstar]
            plsc.store_scatter(scan_v, [_splat_i(jstar)],
                               jnp.ones((16,), jnp.float32), mask=lane0)
            cjs = _gather1(c_v, (r, jstar))
            vjs = _gather1(v_v, (jstar,))
            shift = dmin - cjs + vjs

            def upd_chunk(ci, carry2):
                bv2, bj2 = carry2
                base = ci * 16
                sl = pl.ds(base, 16)
                cand = c_v[r, sl] - v_v[sl] + shift
                d = dist_v[sl]
                sc = scan_v[sl]
                upd = (cand < d) & (sc == 0.0)
                d2 = jnp.where(upd, cand, d)
                dist_v[sl] = d2
                way_v[sl] = jnp.where(upd, _splat_i(jstar), way_v[sl])
                avail = jnp.where(sc == 0.0, d2, _BIGF)
                m = jnp.min(avail)
                mi = jnp.min(jnp.where(avail == m, lane + base,
                                       jnp.int32(1 << 30)))
                better = m < bv2
                return jnp.where(better, m, bv2), jnp.where(better, mi, bj2)

            bv2, bj2 = lax.fori_loop(0, _NCH, upd_chunk,
                                     (jnp.float32(_BIGF), jnp.int32(0)))
            return (bj2, bv2)

        jstar, dmin = lax.while_loop(dcond, dbody, (bj, bv))

        @pl.loop(0, _NCH)
        def _(ci):
            sl = pl.ds(ci * 16, 16)
            sc = scan_v[sl]
            v_v[sl] = jnp.where(sc != 0.0, v_v[sl] + (dist_v[sl] - dmin),
                                v_v[sl])

        def bt_cond(j):
            return _gather1(way_v, (j,)) != -2

        def bt_body(j):
            jprev = _gather1(way_v, (j,))
            r = p_s[jprev]
            p_s[j] = r
            r2c_s[r] = j
            return jprev

        jroot = lax.while_loop(bt_cond, bt_body, jstar)
        p_s[jroot] = i
        r2c_s[i] = jroot
        return _carry

    lax.fori_loop(0, _T, row_phase, jnp.int32(0))

    def fin(rr, acc):
        j = r2c_s[rr]
        cval = _gather1(c_v, (rr, j))
        ce = _gather1(cose_v, (j,))
        return acc + (1.0 + cval) - _EOS * (1.0 - ce)

    s_b = _EOS * s0 + lax.fori_loop(0, _T, fin, jnp.float32(0.0))
    stage_v[...] = jnp.full((16,), s_b, jnp.float32)
    pltpu.async_copy(stage_v, out_hbm.at[wid], sem).wait()


def _sc_solve(cost_pad, cose_pad):
    mesh = plsc.VectorSubcoreMesh(core_axis_name="c", subcore_axis_name="s")
    cp = pltpu.CompilerParams()
    if "needs_layout_passes" in pltpu.CompilerParams.__dataclass_fields__:
        cp = dataclasses.replace(cp, needs_layout_passes=False)
    kern = functools.partial(
        pl.kernel,
        compiler_params=cp,
        out_type=jax.ShapeDtypeStruct((_B, 16), jnp.float32),
        mesh=mesh,
        scratch_types=[
            pltpu.VMEM((_T, _MP), jnp.float32),
            pltpu.VMEM((_MP,), jnp.float32),
            pltpu.VMEM((_MP,), jnp.float32),
            pltpu.VMEM((_MP,), jnp.int32),
            pltpu.VMEM((_MP,), jnp.float32),
            pltpu.VMEM((_MP,), jnp.float32),
            pltpu.VMEM((16,), jnp.float32),
            pltpu.SMEM((_MP,), jnp.int32),
            pltpu.SMEM((_T + 8,), jnp.int32),
            pltpu.SemaphoreType.DMA,
        ],
    )(_sc_body)
    return kern(cost_pad, cose_pad)


def kernel(outputs, targets, empty_vec):
    cost, cose = _tc_cost(outputs, targets, empty_vec)
    cost_p = jnp.pad(cost, ((0, 0), (0, 0), (0, _MP - _Q)),
                     constant_values=_PADC)
    cose_p = jnp.pad(cose[:, 0, :], ((0, 0), (0, _MP - _Q)),
                     constant_values=1.0)
    s = _sc_solve(cost_p, cose_p)
    return jnp.sum(s[:, 0]) / (_B * _Q)


# speedup vs baseline: 74.2665x; 1.3198x over previous
import dataclasses
import functools

import jax
import jax.numpy as jnp
from jax import lax
from jax.experimental import pallas as pl
from jax.experimental.pallas import tpu as pltpu
from jax.experimental.pallas import tpu_sc as plsc

_B = 32
_Q = 300
_T = 40
_D = 512
_MP = 304
_NCH = _MP // 16
_RP = 48
_BPS = 8
_EOS = 0.1
_BIGF = 3e38
_PADC = 1e30


def _tc_body(out_ref, tgt_ref, emp_ref, cx_ref):
    on = out_ref[...].reshape(_BPS * _Q, _D)
    on = on / jnp.maximum(
        jnp.sqrt(jnp.sum(on * on, axis=-1, keepdims=True)), 1e-12)
    tn = tgt_ref[...].reshape(_BPS * _T, _D)
    tn = tn / jnp.maximum(
        jnp.sqrt(jnp.sum(tn * tn, axis=-1, keepdims=True)), 1e-12)
    emp = emp_ref[...]
    en = emp / jnp.maximum(
        jnp.sqrt(jnp.sum(emp * emp, axis=-1, keepdims=True)), 1e-12)

    riota = lax.broadcasted_iota(jnp.int32, (_RP, _MP), 0)
    liota = lax.broadcasted_iota(jnp.int32, (_RP, _MP), 1)
    sign = jnp.where(riota < _T, -1.0, 1.0)
    padv = jnp.where(riota < _T, _PADC, 1.0)
    zrows = jnp.zeros((_RP - _T - 1, _D), jnp.float32)
    zcols = jnp.zeros((_MP - _Q, _D), jnp.float32)

    for k in range(_BPS):
        ext = jnp.concatenate(
            [tn[k * _T:(k + 1) * _T], en, zrows], axis=0)
        onp = jnp.concatenate(
            [on[k * _Q:(k + 1) * _Q], zcols], axis=0)
        full = lax.dot_general(ext, onp, (((1,), (1,)), ((), ())),
                               precision=lax.Precision.HIGHEST,
                               preferred_element_type=jnp.float32)
        cx_ref[k] = jnp.where(liota < _Q, sign * full, padv)


def _tc_cost(outputs, targets, empty_vec):
    return pl.pallas_call(
        _tc_body,
        grid=(_B // _BPS,),
        in_specs=[
            pl.BlockSpec((_BPS, _Q, _D), lambda b: (b, 0, 0)),
            pl.BlockSpec((_BPS, _T, _D), lambda b: (b, 0, 0)),
            pl.BlockSpec((1, _D), lambda b: (0, 0)),
        ],
        out_specs=pl.BlockSpec((_BPS, _RP, _MP), lambda b: (b, 0, 0)),
        out_shape=jax.ShapeDtypeStruct((_B, _RP, _MP), jnp.float32),
        compiler_params=pltpu.CompilerParams(
            dimension_semantics=("arbitrary",)),
    )(outputs, targets, empty_vec)


def _splat_i(x):
    return jnp.full((16,), x, jnp.int32)


def _gather1(ref, idxs):
    return jnp.min(plsc.load_gather(ref, [_splat_i(i) for i in idxs]))


def _sc_body(cx_hbm, out_hbm,
             c_v, v_v, dist_v, way_v, scan_v, stage_v,
             p_s, r2c_s, sem):
    wid = lax.axis_index("s") * 2 + lax.axis_index("c")
    pltpu.async_copy(cx_hbm.at[wid], c_v, sem).wait()

    lane = lax.iota(jnp.int32, 16)
    lane0 = lane == 0

    @pl.loop(0, _MP)
    def _(j):
        p_s[j] = jnp.int32(-1)

    @pl.loop(0, _NCH)
    def _(ci):
        v_v[pl.ds(ci * 16, 16)] = jnp.zeros((16,), jnp.float32)

    def s0_chunk(ci, acc):
        return acc + jnp.sum(1.0 - c_v[_T, pl.ds(ci * 16, 16)])
    s0 = lax.fori_loop(0, _NCH, s0_chunk, jnp.float32(0.0))

    def row_phase(i, _carry):
        def init_chunk(ci, carry):
            bv, bj = carry
            base = ci * 16
            sl = pl.ds(base, 16)
            d = c_v[i, sl] - v_v[sl]
            dist_v[sl] = d
            way_v[sl] = _splat_i(-2)
            scan_v[sl] = jnp.zeros((16,), jnp.float32)
            m = jnp.min(d)
            mi = jnp.min(jnp.where(d == m, lane + base, jnp.int32(1 << 30)))
            better = m < bv
            return jnp.where(better, m, bv), jnp.where(better, mi, bj)

        bv, bj = lax.fori_loop(0, _NCH, init_chunk,
                               (jnp.float32(_BIGF), jnp.int32(0)))

        def dcond(carry):
            jstar, _d = carry
            return p_s[jstar] != -1

        def dbody(carry):
            jstar, dmin = carry
            r = p_s[jstar]
            plsc.store_scatter(scan_v, [_splat_i(jstar)],
                               jnp.ones((16,), jnp.float32), mask=lane0)
            cjs = _gather1(c_v, (r, jstar))
            vjs = _gather1(v_v, (jstar,))
            shift = dmin - cjs + vjs

            def upd_chunk(ci, carry2):
                bv2, bj2 = carry2
                base = ci * 16
                sl = pl.ds(base, 16)
                cand = c_v[r, sl] - v_v[sl] + shift
                d = dist_v[sl]
                sc = scan_v[sl]
                upd = (cand < d) & (sc == 0.0)
                d2 = jnp.where(upd, cand, d)
                dist_v[sl] = d2
                way_v[sl] = jnp.where(upd, _splat_i(jstar), way_v[sl])
                avail = jnp.where(sc == 0.0, d2, _BIGF)
                m = jnp.min(avail)
                mi = jnp.min(jnp.where(avail == m, lane + base,
                                       jnp.int32(1 << 30)))
                better = m < bv2
                return jnp.where(better, m, bv2), jnp.where(better, mi, bj2)

            bv2, bj2 = lax.fori_loop(0, _NCH, upd_chunk,
                                     (jnp.float32(_BIGF), jnp.int32(0)))
            return (bj2, bv2)

        jstar, dmin = lax.while_loop(dcond, dbody, (bj, bv))

        @pl.loop(0, _NCH)
        def _(ci):
            sl = pl.ds(ci * 16, 16)
            sc = scan_v[sl]
            v_v[sl] = jnp.where(sc != 0.0, v_v[sl] + (dist_v[sl] - dmin),
                                v_v[sl])

        def bt_cond(j):
            return _gather1(way_v, (j,)) != -2

        def bt_body(j):
            jprev = _gather1(way_v, (j,))
            r = p_s[jprev]
            p_s[j] = r
            r2c_s[r] = j
            return jprev

        jroot = lax.while_loop(bt_cond, bt_body, jstar)
        p_s[jroot] = i
        r2c_s[i] = jroot
        return _carry

    lax.fori_loop(0, _T, row_phase, jnp.int32(0))

    def fin(rr, acc):
        j = r2c_s[rr]
        cval = _gather1(c_v, (rr, j))
        ce = _gather1(c_v, (_T, j))
        return acc + (1.0 + cval) - _EOS * (1.0 - ce)

    s_b = _EOS * s0 + lax.fori_loop(0, _T, fin, jnp.float32(0.0))
    stage_v[...] = jnp.full((16,), s_b, jnp.float32)
    pltpu.async_copy(stage_v, out_hbm.at[wid], sem).wait()


def _sc_solve(cx):
    mesh = plsc.VectorSubcoreMesh(core_axis_name="c", subcore_axis_name="s")
    cp = pltpu.CompilerParams()
    if "needs_layout_passes" in pltpu.CompilerParams.__dataclass_fields__:
        cp = dataclasses.replace(cp, needs_layout_passes=False)
    kern = functools.partial(
        pl.kernel,
        compiler_params=cp,
        out_type=jax.ShapeDtypeStruct((_B, 16), jnp.float32),
        mesh=mesh,
        scratch_types=[
            pltpu.VMEM((_RP, _MP), jnp.float32),
            pltpu.VMEM((_MP,), jnp.float32),
            pltpu.VMEM((_MP,), jnp.float32),
            pltpu.VMEM((_MP,), jnp.int32),
            pltpu.VMEM((_MP,), jnp.float32),
            pltpu.VMEM((16,), jnp.float32),
            pltpu.SMEM((_MP,), jnp.int32),
            pltpu.SMEM((_T + 8,), jnp.int32),
            pltpu.SemaphoreType.DMA,
        ],
    )(_sc_body)
    return kern(cx)


def kernel(outputs, targets, empty_vec):
    cx = _tc_cost(outputs, targets, empty_vec)
    s = _sc_solve(cx)
    return jnp.sum(s[:, 0]) / (_B * _Q)


# speedup vs baseline: 85.6161x; 1.1528x over previous
import dataclasses
import functools

import jax
import jax.numpy as jnp
from jax import lax
from jax.experimental import pallas as pl
from jax.experimental.pallas import tpu as pltpu
from jax.experimental.pallas import tpu_sc as plsc

_B = 32
_Q = 300
_T = 40
_D = 512
_MP = 304
_NCH = _MP // 16
_RP = 48
_BPS = 8
_EOS = 0.1
_BIGF = 3e38
_PADC = 1e30


def _tc_body(out_ref, tgt_ref, emp_ref, cx_ref):
    on = out_ref[...].reshape(_BPS * _Q, _D)
    on = on * lax.rsqrt(
        jnp.maximum(jnp.sum(on * on, axis=-1, keepdims=True), 1e-24))
    tn = tgt_ref[...].reshape(_BPS * _T, _D)
    tn = tn * lax.rsqrt(
        jnp.maximum(jnp.sum(tn * tn, axis=-1, keepdims=True), 1e-24))
    emp = emp_ref[...]
    en = emp * lax.rsqrt(
        jnp.maximum(jnp.sum(emp * emp, axis=-1, keepdims=True), 1e-24))
    on = on.astype(jnp.bfloat16)
    tn = tn.astype(jnp.bfloat16)
    en = en.astype(jnp.bfloat16)

    priota = lax.broadcasted_iota(jnp.int32, (_RP, _MP - _Q), 0)
    padv = jnp.where(priota < _T, _PADC, 1.0)
    dn = (((1,), (1,)), ((), ()))

    for k in range(_BPS):
        onk = on[k * _Q:(k + 1) * _Q]
        f40 = lax.dot_general(tn[k * _T:(k + 1) * _T], onk, dn,
                              preferred_element_type=jnp.float32)
        fe = lax.dot_general(en, onk, dn,
                             preferred_element_type=jnp.float32)
        cx_ref[k, 0:_T, 0:_Q] = -f40
        cx_ref[k, _T:_T + 1, 0:_Q] = fe
        cx_ref[k, :, _Q:_MP] = padv


def _tc_cost(outputs, targets, empty_vec):
    return pl.pallas_call(
        _tc_body,
        grid=(_B // _BPS,),
        in_specs=[
            pl.BlockSpec((_BPS, _Q, _D), lambda b: (b, 0, 0)),
            pl.BlockSpec((_BPS, _T, _D), lambda b: (b, 0, 0)),
            pl.BlockSpec((1, _D), lambda b: (0, 0)),
        ],
        out_specs=pl.BlockSpec((_BPS, _RP, _MP), lambda b: (b, 0, 0)),
        out_shape=jax.ShapeDtypeStruct((_B, _RP, _MP), jnp.float32),
        compiler_params=pltpu.CompilerParams(
            dimension_semantics=("arbitrary",)),
    )(outputs, targets, empty_vec)


def _splat_i(x):
    return jnp.full((16,), x, jnp.int32)


def _gather1(ref, idxs):
    return jnp.min(plsc.load_gather(ref, [_splat_i(i) for i in idxs]))


def _sc_body(cx_hbm, out_hbm,
             c_v, v_v, dist_v, way_v, scan_v, stage_v,
             p_s, r2c_s, sem):
    wid = lax.axis_index("s") * 2 + lax.axis_index("c")
    pltpu.async_copy(cx_hbm.at[wid], c_v, sem).wait()

    lane = lax.iota(jnp.int32, 16)
    lane0 = lane == 0

    @pl.loop(0, _MP)
    def _(j):
        p_s[j] = jnp.int32(-1)

    @pl.loop(0, _NCH)
    def _(ci):
        v_v[pl.ds(ci * 16, 16)] = jnp.zeros((16,), jnp.float32)

    def s0_chunk(ci, acc):
        return acc + jnp.sum(1.0 - c_v[_T, pl.ds(ci * 16, 16)])
    s0 = lax.fori_loop(0, _NCH, s0_chunk, jnp.float32(0.0))

    def row_phase(i, _carry):
        def init_chunk(ci, carry):
            bv, bj = carry
            base = ci * 16
            sl = pl.ds(base, 16)
            d = c_v[i, sl] - v_v[sl]
            dist_v[sl] = d
            way_v[sl] = _splat_i(-2)
            scan_v[sl] = jnp.zeros((16,), jnp.float32)
            m = jnp.min(d)
            mi = jnp.min(jnp.where(d == m, lane + base, jnp.int32(1 << 30)))
            better = m < bv
            return jnp.where(better, m, bv), jnp.where(better, mi, bj)

        bv, bj = lax.fori_loop(0, _NCH, init_chunk,
                               (jnp.float32(_BIGF), jnp.int32(0)))

        def dcond(carry):
            jstar, _d = carry
            return p_s[jstar] != -1

        def dbody(carry):
            jstar, dmin = carry
            r = p_s[jstar]
            plsc.store_scatter(scan_v, [_splat_i(jstar)],
                               jnp.ones((16,), jnp.float32), mask=lane0)
            cjs = _gather1(c_v, (r, jstar))
            vjs = _gather1(v_v, (jstar,))
            shift = dmin - cjs + vjs

            def upd_chunk(ci, carry2):
                bv2, bj2 = carry2
                base = ci * 16
                sl = pl.ds(base, 16)
                cand = c_v[r, sl] - v_v[sl] + shift
                d = dist_v[sl]
                sc = scan_v[sl]
                upd = (cand < d) & (sc == 0.0)
                d2 = jnp.where(upd, cand, d)
                dist_v[sl] = d2
                way_v[sl] = jnp.where(upd, _splat_i(jstar), way_v[sl])
                avail = jnp.where(sc == 0.0, d2, _BIGF)
                m = jnp.min(avail)
                mi = jnp.min(jnp.where(avail == m, lane + base,
                                       jnp.int32(1 << 30)))
                better = m < bv2
                return jnp.where(better, m, bv2), jnp.where(better, mi, bj2)

            bv2, bj2 = lax.fori_loop(0, _NCH, upd_chunk,
                                     (jnp.float32(_BIGF), jnp.int32(0)))
            return (bj2, bv2)

        jstar, dmin = lax.while_loop(dcond, dbody, (bj, bv))

        @pl.loop(0, _NCH)
        def _(ci):
            sl = pl.ds(ci * 16, 16)
            sc = scan_v[sl]
            v_v[sl] = jnp.where(sc != 0.0, v_v[sl] + (dist_v[sl] - dmin),
                                v_v[sl])

        def bt_cond(j):
            return _gather1(way_v, (j,)) != -2

        def bt_body(j):
            jprev = _gather1(way_v, (j,))
            r = p_s[jprev]
            p_s[j] = r
            r2c_s[r] = j
            return jprev

        jroot = lax.while_loop(bt_cond, bt_body, jstar)
        p_s[jroot] = i
        r2c_s[i] = jroot
        return _carry

    lax.fori_loop(0, _T, row_phase, jnp.int32(0))

    def fin(rr, acc):
        j = r2c_s[rr]
        cval = _gather1(c_v, (rr, j))
        ce = _gather1(c_v, (_T, j))
        return acc + (1.0 + cval) - _EOS * (1.0 - ce)

    s_b = _EOS * s0 + lax.fori_loop(0, _T, fin, jnp.float32(0.0))
    stage_v[...] = jnp.full((16,), s_b, jnp.float32)
    pltpu.async_copy(stage_v, out_hbm.at[wid], sem).wait()


def _sc_solve(cx):
    mesh = plsc.VectorSubcoreMesh(core_axis_name="c", subcore_axis_name="s")
    cp = pltpu.CompilerParams()
    if "needs_layout_passes" in pltpu.CompilerParams.__dataclass_fields__:
        cp = dataclasses.replace(cp, needs_layout_passes=False)
    kern = functools.partial(
        pl.kernel,
        compiler_params=cp,
        out_type=jax.ShapeDtypeStruct((_B, 16), jnp.float32),
        mesh=mesh,
        scratch_types=[
            pltpu.VMEM((_RP, _MP), jnp.float32),
            pltpu.VMEM((_MP,), jnp.float32),
            pltpu.VMEM((_MP,), jnp.float32),
            pltpu.VMEM((_MP,), jnp.int32),
            pltpu.VMEM((_MP,), jnp.float32),
            pltpu.VMEM((16,), jnp.float32),
            pltpu.SMEM((_MP,), jnp.int32),
            pltpu.SMEM((_T + 8,), jnp.int32),
            pltpu.SemaphoreType.DMA,
        ],
    )(_sc_body)
    return kern(cx)


def kernel(outputs, targets, empty_vec):
    cx = _tc_cost(outputs, targets, empty_vec)
    s = _sc_solve(cx)
    return jnp.sum(s[:, 0]) / (_B * _Q)


# speedup vs baseline: 92.6286x; 1.0819x over previous
import dataclasses
import functools

import jax
import jax.numpy as jnp
from jax import lax
from jax.experimental import pallas as pl
from jax.experimental.pallas import tpu as pltpu
from jax.experimental.pallas import tpu_sc as plsc

_B = 32
_Q = 300
_T = 40
_D = 512
_MP = 304
_NCH = _MP // 16
_RP = 48
_BPS = 16
_EOS = 0.1
_BIGF = 3e38
_PADC = 1e30


def _tc_body(out_ref, tgt_ref, emp_ref, cx_ref):
    on = out_ref[...].reshape(_BPS * _Q, _D)
    on = on * lax.rsqrt(
        jnp.maximum(jnp.sum(on * on, axis=-1, keepdims=True), 1e-24))
    tn = tgt_ref[...].reshape(_BPS * _T, _D)
    tn = tn * lax.rsqrt(
        jnp.maximum(jnp.sum(tn * tn, axis=-1, keepdims=True), 1e-24))
    emp = emp_ref[...]
    en = emp * lax.rsqrt(
        jnp.maximum(jnp.sum(emp * emp, axis=-1, keepdims=True), 1e-24))
    on = on.astype(jnp.bfloat16)
    tn = tn.astype(jnp.bfloat16)
    en = en.astype(jnp.bfloat16)

    priota = lax.broadcasted_iota(jnp.int32, (_RP, _MP - _Q), 0)
    padv = jnp.where(priota < _T, _PADC, 1.0)
    dn = (((1,), (1,)), ((), ()))

    for k in range(_BPS):
        onk = on[k * _Q:(k + 1) * _Q]
        f40 = lax.dot_general(tn[k * _T:(k + 1) * _T], onk, dn,
                              preferred_element_type=jnp.float32)
        fe = lax.dot_general(en, onk, dn,
                             preferred_element_type=jnp.float32)
        cx_ref[k, 0:_T, 0:_Q] = -f40
        cx_ref[k, _T:_T + 1, 0:_Q] = fe
        cx_ref[k, :, _Q:_MP] = padv


def _tc_cost(outputs, targets, empty_vec):
    return pl.pallas_call(
        _tc_body,
        grid=(_B // _BPS,),
        in_specs=[
            pl.BlockSpec((_BPS, _Q, _D), lambda b: (b, 0, 0)),
            pl.BlockSpec((_BPS, _T, _D), lambda b: (b, 0, 0)),
            pl.BlockSpec((1, _D), lambda b: (0, 0)),
        ],
        out_specs=pl.BlockSpec((_BPS, _RP, _MP), lambda b: (b, 0, 0)),
        out_shape=jax.ShapeDtypeStruct((_B, _RP, _MP), jnp.float32),
        compiler_params=pltpu.CompilerParams(
            dimension_semantics=("arbitrary",)),
    )(outputs, targets, empty_vec)


def _splat_i(x):
    return jnp.full((16,), x, jnp.int32)


def _gather1(ref, idxs):
    return jnp.min(plsc.load_gather(ref, [_splat_i(i) for i in idxs]))


def _sc_body(cx_hbm, out_hbm,
             c_v, v_v, dist_v, way_v, scan_v, stage_v,
             p_s, r2c_s, sem):
    wid = lax.axis_index("s") * 2 + lax.axis_index("c")
    pltpu.async_copy(cx_hbm.at[wid], c_v, sem).wait()

    lane = lax.iota(jnp.int32, 16)
    lane0 = lane == 0

    @pl.loop(0, _MP, step=8)
    def _(j):
        for jj in range(8):
            p_s[j + jj] = jnp.int32(-1)

    for ci in range(_NCH):
        v_v[pl.ds(ci * 16, 16)] = jnp.zeros((16,), jnp.float32)

    s0 = jnp.float32(0.0)
    for ci in range(_NCH):
        s0 = s0 + jnp.sum(1.0 - c_v[_T, pl.ds(ci * 16, 16)])

    def row_phase(i, dmin_prev):
        first = i == 0
        bv, bj = jnp.float32(_BIGF), jnp.int32(0)
        for ci in range(_NCH):
            base = ci * 16
            sl = pl.ds(base, 16)
            sc = scan_v[sl]
            vv = v_v[sl]
            vup = (sc != 0.0) & jnp.logical_not(
                jnp.broadcast_to(first, (16,)))
            vv = jnp.where(vup, vv + (dist_v[sl] - dmin_prev), vv)
            v_v[sl] = vv
            d = c_v[i, sl] - vv
            dist_v[sl] = d
            way_v[sl] = _splat_i(-2)
            scan_v[sl] = jnp.zeros((16,), jnp.float32)
            m = jnp.min(d)
            mi = jnp.min(jnp.where(d == m, lane + base, jnp.int32(1 << 30)))
            better = m < bv
            bv = jnp.where(better, m, bv)
            bj = jnp.where(better, mi, bj)

        def dcond(carry):
            jstar, _d = carry
            return p_s[jstar] != -1

        def dbody(carry):
            jstar, dmin = carry
            r = p_s[jstar]
            plsc.store_scatter(scan_v, [_splat_i(jstar)],
                               jnp.ones((16,), jnp.float32), mask=lane0)
            cjs = _gather1(c_v, (r, jstar))
            vjs = _gather1(v_v, (jstar,))
            shift = dmin - cjs + vjs

            bv2, bj2 = jnp.float32(_BIGF), jnp.int32(0)
            for ci in range(_NCH):
                base = ci * 16
                sl = pl.ds(base, 16)
                cand = c_v[r, sl] - v_v[sl] + shift
                d = dist_v[sl]
                sc = scan_v[sl]
                upd = (cand < d) & (sc == 0.0)
                d2 = jnp.where(upd, cand, d)
                dist_v[sl] = d2
                way_v[sl] = jnp.where(upd, _splat_i(jstar), way_v[sl])
                avail = jnp.where(sc == 0.0, d2, _BIGF)
                m = jnp.min(avail)
                mi = jnp.min(jnp.where(avail == m, lane + base,
                                       jnp.int32(1 << 30)))
                better = m < bv2
                bv2 = jnp.where(better, m, bv2)
                bj2 = jnp.where(better, mi, bj2)
            return (bj2, bv2)

        jstar, dmin = lax.while_loop(dcond, dbody, (bj, bv))

        def bt_cond(j):
            return _gather1(way_v, (j,)) != -2

        def bt_body(j):
            jprev = _gather1(way_v, (j,))
            r = p_s[jprev]
            p_s[j] = r
            r2c_s[r] = j
            return jprev

        jroot = lax.while_loop(bt_cond, bt_body, jstar)
        p_s[jroot] = i
        r2c_s[i] = jroot
        return dmin

    lax.fori_loop(0, _T, row_phase, jnp.float32(0.0))

    def fin(rr, acc):
        j = r2c_s[rr]
        cval = _gather1(c_v, (rr, j))
        ce = _gather1(c_v, (_T, j))
        return acc + (1.0 + cval) - _EOS * (1.0 - ce)

    s_b = _EOS * s0 + lax.fori_loop(0, _T, fin, jnp.float32(0.0))
    stage_v[...] = jnp.full((16,), s_b, jnp.float32)
    pltpu.async_copy(stage_v, out_hbm.at[wid], sem).wait()


def _sc_solve(cx):
    mesh = plsc.VectorSubcoreMesh(core_axis_name="c", subcore_axis_name="s")
    cp = pltpu.CompilerParams()
    if "needs_layout_passes" in pltpu.CompilerParams.__dataclass_fields__:
        cp = dataclasses.replace(cp, needs_layout_passes=False)
    kern = functools.partial(
        pl.kernel,
        compiler_params=cp,
        out_type=jax.ShapeDtypeStruct((_B, 16), jnp.float32),
        mesh=mesh,
        scratch_types=[
            pltpu.VMEM((_RP, _MP), jnp.float32),
            pltpu.VMEM((_MP,), jnp.float32),
            pltpu.VMEM((_MP,), jnp.float32),
            pltpu.VMEM((_MP,), jnp.int32),
            pltpu.VMEM((_MP,), jnp.float32),
            pltpu.VMEM((16,), jnp.float32),
            pltpu.SMEM((_MP,), jnp.int32),
            pltpu.SMEM((_T + 8,), jnp.int32),
            pltpu.SemaphoreType.DMA,
        ],
    )(_sc_body)
    return kern(cx)


def kernel(outputs, targets, empty_vec):
    cx = _tc_cost(outputs, targets, empty_vec)
    s = _sc_solve(cx)
    return jnp.sum(s[:, 0]) / (_B * _Q)


# speedup vs baseline: 94.9154x; 1.0247x over previous
import dataclasses
import functools

import jax
import jax.numpy as jnp
from jax import lax
from jax.experimental import pallas as pl
from jax.experimental.pallas import tpu as pltpu
from jax.experimental.pallas import tpu_sc as plsc

_B = 32
_Q = 300
_T = 40
_D = 512
_MP = 304
_NCH = _MP // 16
_RP = 48
_BPS = 16
_EOS = 0.1
_BIGF = 3e38
_PADC = 1e30


def _tc_body(out_ref, tgt_ref, emp_ref, cx_ref):
    on = out_ref[...].reshape(_BPS * _Q, _D)
    on = on * lax.rsqrt(
        jnp.maximum(jnp.sum(on * on, axis=-1, keepdims=True), 1e-24))
    tn = tgt_ref[...].reshape(_BPS * _T, _D)
    tn = tn * lax.rsqrt(
        jnp.maximum(jnp.sum(tn * tn, axis=-1, keepdims=True), 1e-24))
    emp = emp_ref[...]
    en = emp * lax.rsqrt(
        jnp.maximum(jnp.sum(emp * emp, axis=-1, keepdims=True), 1e-24))
    on = on.astype(jnp.bfloat16)
    tn = tn.astype(jnp.bfloat16)
    en = en.astype(jnp.bfloat16)

    priota = lax.broadcasted_iota(jnp.int32, (_RP, _MP - _Q), 0)
    padv = jnp.where(priota < _T, _PADC, 1.0)
    dn = (((1,), (1,)), ((), ()))

    for k in range(_BPS):
        onk = on[k * _Q:(k + 1) * _Q]
        f40 = lax.dot_general(tn[k * _T:(k + 1) * _T], onk, dn,
                              preferred_element_type=jnp.float32)
        fe = lax.dot_general(en, onk, dn,
                             preferred_element_type=jnp.float32)
        cx_ref[k, 0:_T, 0:_Q] = -f40
        cx_ref[k, _T:_T + 1, 0:_Q] = fe
        cx_ref[k, :, _Q:_MP] = padv


def _tc_cost(outputs, targets, empty_vec):
    return pl.pallas_call(
        _tc_body,
        grid=(_B // _BPS,),
        in_specs=[
            pl.BlockSpec((_BPS, _Q, _D), lambda b: (b, 0, 0)),
            pl.BlockSpec((_BPS, _T, _D), lambda b: (b, 0, 0)),
            pl.BlockSpec((1, _D), lambda b: (0, 0)),
        ],
        out_specs=pl.BlockSpec((_BPS, _RP, _MP), lambda b: (b, 0, 0)),
        out_shape=jax.ShapeDtypeStruct((_B, _RP, _MP), jnp.float32),
        compiler_params=pltpu.CompilerParams(
            dimension_semantics=("arbitrary",)),
    )(outputs, targets, empty_vec)


def _splat_i(x):
    return jnp.full((16,), x, jnp.int32)


def _gather1(ref, idxs):
    return jnp.min(plsc.load_gather(ref, [_splat_i(i) for i in idxs]))


def _sc_body(cx_hbm, out_hbm,
             c_v, v_v, dist_v, way_v, scan_v, stage_v,
             p_s, r2c_s, sem):
    wid = lax.axis_index("s") * 2 + lax.axis_index("c")
    pltpu.async_copy(cx_hbm.at[wid], c_v, sem).wait()

    lane = lax.iota(jnp.int32, 16)
    lane0 = lane == 0

    @pl.loop(0, _MP, step=8)
    def _(j):
        for jj in range(8):
            p_s[j + jj] = jnp.int32(-1)

    for ci in range(_NCH):
        v_v[pl.ds(ci * 16, 16)] = jnp.zeros((16,), jnp.float32)

    s0 = jnp.float32(0.0)
    for ci in range(_NCH):
        s0 = s0 + jnp.sum(1.0 - c_v[_T, pl.ds(ci * 16, 16)])

    def row_phase(i, dmin_prev):
        first = i == 0
        vmin = jnp.full((16,), _BIGF, jnp.float32)
        vidx = jnp.zeros((16,), jnp.int32)
        for ci in range(_NCH):
            base = ci * 16
            sl = pl.ds(base, 16)
            sc = scan_v[sl]
            vv = v_v[sl]
            vup = (sc != 0.0) & jnp.logical_not(
                jnp.broadcast_to(first, (16,)))
            vv = jnp.where(vup, vv + (dist_v[sl] - dmin_prev), vv)
            v_v[sl] = vv
            d = c_v[i, sl] - vv
            dist_v[sl] = d
            way_v[sl] = _splat_i(-2)
            scan_v[sl] = jnp.zeros((16,), jnp.float32)
            cmp = d < vmin
            vidx = jnp.where(cmp, lane + base, vidx)
            vmin = jnp.where(cmp, d, vmin)
        bv = jnp.min(vmin)
        bj = jnp.min(jnp.where(vmin == bv, vidx, jnp.int32(1 << 30)))

        def dcond(carry):
            jstar, _d = carry
            return p_s[jstar] != -1

        def dbody(carry):
            jstar, dmin = carry
            r = p_s[jstar]
            plsc.store_scatter(scan_v, [_splat_i(jstar)],
                               jnp.ones((16,), jnp.float32), mask=lane0)
            cjs = _gather1(c_v, (r, jstar))
            vjs = _gather1(v_v, (jstar,))
            shift = dmin - cjs + vjs

            vmin = jnp.full((16,), _BIGF, jnp.float32)
            vidx = jnp.zeros((16,), jnp.int32)
            for ci in range(_NCH):
                base = ci * 16
                sl = pl.ds(base, 16)
                cand = c_v[r, sl] - v_v[sl] + shift
                d = dist_v[sl]
                sc = scan_v[sl]
                upd = (cand < d) & (sc == 0.0)
                d2 = jnp.where(upd, cand, d)
                dist_v[sl] = d2
                way_v[sl] = jnp.where(upd, _splat_i(jstar), way_v[sl])
                avail = jnp.where(sc == 0.0, d2, _BIGF)
                cmp = avail < vmin
                vidx = jnp.where(cmp, lane + base, vidx)
                vmin = jnp.where(cmp, avail, vmin)
            bv2 = jnp.min(vmin)
            bj2 = jnp.min(jnp.where(vmin == bv2, vidx, jnp.int32(1 << 30)))
            return (bj2, bv2)

        jstar, dmin = lax.while_loop(dcond, dbody, (bj, bv))

        def bt_cond(j):
            return _gather1(way_v, (j,)) != -2

        def bt_body(j):
            jprev = _gather1(way_v, (j,))
            r = p_s[jprev]
            p_s[j] = r
            r2c_s[r] = j
            return jprev

        jroot = lax.while_loop(bt_cond, bt_body, jstar)
        p_s[jroot] = i
        r2c_s[i] = jroot
        return dmin

    lax.fori_loop(0, _T, row_phase, jnp.float32(0.0))

    def fin(rr, acc):
        j = r2c_s[rr]
        cval = _gather1(c_v, (rr, j))
        ce = _gather1(c_v, (_T, j))
        return acc + (1.0 + cval) - _EOS * (1.0 - ce)

    s_b = _EOS * s0 + lax.fori_loop(0, _T, fin, jnp.float32(0.0))
    stage_v[...] = jnp.full((16,), s_b, jnp.float32)
    pltpu.async_copy(stage_v, out_hbm.at[wid], sem).wait()


def _sc_solve(cx):
    mesh = plsc.VectorSubcoreMesh(core_axis_name="c", subcore_axis_name="s")
    cp = pltpu.CompilerParams()
    if "needs_layout_passes" in pltpu.CompilerParams.__dataclass_fields__:
        cp = dataclasses.replace(cp, needs_layout_passes=False)
    kern = functools.partial(
        pl.kernel,
        compiler_params=cp,
        out_type=jax.ShapeDtypeStruct((_B, 16), jnp.float32),
        mesh=mesh,
        scratch_types=[
            pltpu.VMEM((_RP, _MP), jnp.float32),
            pltpu.VMEM((_MP,), jnp.float32),
            pltpu.VMEM((_MP,), jnp.float32),
            pltpu.VMEM((_MP,), jnp.int32),
            pltpu.VMEM((_MP,), jnp.float32),
            pltpu.VMEM((16,), jnp.float32),
            pltpu.SMEM((_MP,), jnp.int32),
            pltpu.SMEM((_T + 8,), jnp.int32),
            pltpu.SemaphoreType.DMA,
        ],
    )(_sc_body)
    return kern(cx)


def kernel(outputs, targets, empty_vec):
    cx = _tc_cost(outputs, targets, empty_vec)
    s = _sc_solve(cx)
    return jnp.sum(s[:, 0]) / (_B * _Q)


# speedup vs baseline: 95.3566x; 1.0046x over previous
import dataclasses
import functools

import jax
import jax.numpy as jnp
from jax import lax
from jax.experimental import pallas as pl
from jax.experimental.pallas import tpu as pltpu
from jax.experimental.pallas import tpu_sc as plsc

_B = 32
_Q = 300
_T = 40
_D = 512
_MP = 304
_NCH = _MP // 16
_RP = 48
_BPS = 8
_EOS = 0.1
_BIGF = 3e38
_PADC = 1e30


def _tc_body(out_ref, tgt_ref, emp_ref, cx_ref):
    on = out_ref[...].reshape(_BPS * _Q, _D)
    on = on * lax.rsqrt(
        jnp.maximum(jnp.sum(on * on, axis=-1, keepdims=True), 1e-24))
    tn = tgt_ref[...].reshape(_BPS * _T, _D)
    tn = tn * lax.rsqrt(
        jnp.maximum(jnp.sum(tn * tn, axis=-1, keepdims=True), 1e-24))
    emp = emp_ref[...]
    en = emp * lax.rsqrt(
        jnp.maximum(jnp.sum(emp * emp, axis=-1, keepdims=True), 1e-24))
    on = on.astype(jnp.bfloat16)
    tn = tn.astype(jnp.bfloat16)
    en = en.astype(jnp.bfloat16)

    priota = lax.broadcasted_iota(jnp.int32, (_RP, _MP - _Q), 0)
    padv = jnp.where(priota < _T, _PADC, 1.0)
    dn = (((1,), (1,)), ((), ()))

    for k in range(_BPS):
        onk = on[k * _Q:(k + 1) * _Q]
        f40 = lax.dot_general(tn[k * _T:(k + 1) * _T], onk, dn,
                              preferred_element_type=jnp.float32)
        fe = lax.dot_general(en, onk, dn,
                             preferred_element_type=jnp.float32)
        cx_ref[k, 0:_T, 0:_Q] = -f40
        cx_ref[k, _T:_T + 1, 0:_Q] = fe
        cx_ref[k, :, _Q:_MP] = padv


def _tc_cost(outputs, targets, empty_vec):
    return pl.pallas_call(
        _tc_body,
        grid=(_B // _BPS,),
        in_specs=[
            pl.BlockSpec((_BPS, _Q, _D), lambda b: (b, 0, 0)),
            pl.BlockSpec((_BPS, _T, _D), lambda b: (b, 0, 0)),
            pl.BlockSpec((1, _D), lambda b: (0, 0)),
        ],
        out_specs=pl.BlockSpec((_BPS, _RP, _MP), lambda b: (b, 0, 0)),
        out_shape=jax.ShapeDtypeStruct((_B, _RP, _MP), jnp.float32),
        compiler_params=pltpu.CompilerParams(
            dimension_semantics=("parallel",)),
    )(outputs, targets, empty_vec)


def _splat_i(x):
    return jnp.full((16,), x, jnp.int32)


def _gather1(ref, idxs):
    return jnp.min(plsc.load_gather(ref, [_splat_i(i) for i in idxs]))


def _sc_body(cx_hbm, out_hbm,
             c_v, v_v, dist_v, way_v, scan_v, stage_v,
             p_s, r2c_s, sem):
    wid = lax.axis_index("s") * 2 + lax.axis_index("c")
    pltpu.async_copy(cx_hbm.at[wid], c_v, sem).wait()

    lane = lax.iota(jnp.int32, 16)
    lane0 = lane == 0

    @pl.loop(0, _MP, step=8)
    def _(j):
        for jj in range(8):
            p_s[j + jj] = jnp.int32(-1)

    for ci in range(_NCH):
        v_v[pl.ds(ci * 16, 16)] = jnp.zeros((16,), jnp.float32)

    s0 = jnp.float32(0.0)
    for ci in range(_NCH):
        s0 = s0 + jnp.sum(1.0 - c_v[_T, pl.ds(ci * 16, 16)])

    def row_phase(i, dmin_prev):
        first = i == 0
        vmin = jnp.full((16,), _BIGF, jnp.float32)
        vidx = jnp.zeros((16,), jnp.int32)
        for ci in range(_NCH):
            base = ci * 16
            sl = pl.ds(base, 16)
            sc = scan_v[sl]
            vv = v_v[sl]
            vup = (sc != 0.0) & jnp.logical_not(
                jnp.broadcast_to(first, (16,)))
            vv = jnp.where(vup, vv + (dist_v[sl] - dmin_prev), vv)
            v_v[sl] = vv
            d = c_v[i, sl] - vv
            dist_v[sl] = d
            way_v[sl] = _splat_i(-2)
            scan_v[sl] = jnp.zeros((16,), jnp.float32)
            cmp = d < vmin
            vidx = jnp.where(cmp, lane + base, vidx)
            vmin = jnp.where(cmp, d, vmin)
        bv = jnp.min(vmin)
        bj = jnp.min(jnp.where(vmin == bv, vidx, jnp.int32(1 << 30)))

        def dcond(carry):
            jstar, _d = carry
            return p_s[jstar] != -1

        def dbody(carry):
            jstar, dmin = carry
            r = p_s[jstar]
            plsc.store_scatter(scan_v, [_splat_i(jstar)],
                               jnp.ones((16,), jnp.float32), mask=lane0)
            cjs = _gather1(c_v, (r, jstar))
            vjs = _gather1(v_v, (jstar,))
            shift = dmin - cjs + vjs

            vmin = jnp.full((16,), _BIGF, jnp.float32)
            vidx = jnp.zeros((16,), jnp.int32)
            for ci in range(_NCH):
                base = ci * 16
                sl = pl.ds(base, 16)
                cand = c_v[r, sl] - v_v[sl] + shift
                d = dist_v[sl]
                sc = scan_v[sl]
                upd = (cand < d) & (sc == 0.0)
                d2 = jnp.where(upd, cand, d)
                dist_v[sl] = d2
                way_v[sl] = jnp.where(upd, _splat_i(jstar), way_v[sl])
                avail = jnp.where(sc == 0.0, d2, _BIGF)
                cmp = avail < vmin
                vidx = jnp.where(cmp, lane + base, vidx)
                vmin = jnp.where(cmp, avail, vmin)
            bv2 = jnp.min(vmin)
            bj2 = jnp.min(jnp.where(vmin == bv2, vidx, jnp.int32(1 << 30)))
            return (bj2, bv2)

        jstar, dmin = lax.while_loop(dcond, dbody, (bj, bv))

        def bt_cond(j):
            return _gather1(way_v, (j,)) != -2

        def bt_body(j):
            jprev = _gather1(way_v, (j,))
            r = p_s[jprev]
            p_s[j] = r
            r2c_s[r] = j
            return jprev

        jroot = lax.while_loop(bt_cond, bt_body, jstar)
        p_s[jroot] = i
        r2c_s[i] = jroot
        return dmin

    lax.fori_loop(0, _T, row_phase, jnp.float32(0.0))

    def fin(rr, acc):
        j = r2c_s[rr]
        cval = _gather1(c_v, (rr, j))
        ce = _gather1(c_v, (_T, j))
        return acc + (1.0 + cval) - _EOS * (1.0 - ce)

    s_b = _EOS * s0 + lax.fori_loop(0, _T, fin, jnp.float32(0.0))
    stage_v[...] = jnp.full((16,), s_b, jnp.float32)
    pltpu.async_copy(stage_v, out_hbm.at[wid], sem).wait()


def _sc_solve(cx):
    mesh = plsc.VectorSubcoreMesh(core_axis_name="c", subcore_axis_name="s")
    cp = pltpu.CompilerParams()
    if "needs_layout_passes" in pltpu.CompilerParams.__dataclass_fields__:
        cp = dataclasses.replace(cp, needs_layout_passes=False)
    kern = functools.partial(
        pl.kernel,
        compiler_params=cp,
        out_type=jax.ShapeDtypeStruct((_B, 16), jnp.float32),
        mesh=mesh,
        scratch_types=[
            pltpu.VMEM((_RP, _MP), jnp.float32),
            pltpu.VMEM((_MP,), jnp.float32),
            pltpu.VMEM((_MP,), jnp.float32),
            pltpu.VMEM((_MP,), jnp.int32),
            pltpu.VMEM((_MP,), jnp.float32),
            pltpu.VMEM((16,), jnp.float32),
            pltpu.SMEM((_MP,), jnp.int32),
            pltpu.SMEM((_T + 8,), jnp.int32),
            pltpu.SemaphoreType.DMA,
        ],
    )(_sc_body)
    return kern(cx)


def kernel(outputs, targets, empty_vec):
    cx = _tc_cost(outputs, targets, empty_vec)
    s = _sc_solve(cx)
    return jnp.sum(s[:, 0]) / (_B * _Q)
